# baseline (speedup 1.0000x reference)
import functools

import jax
import jax.numpy as jnp
from jax.experimental import pallas as pl
from jax.experimental.pallas import tpu as pltpu
from jax.experimental.pallas import tpu_sc as plsc

_PREC = jax.lax.Precision.DEFAULT


def _pick(n, pref):
    for c in (pref, 512, 256, 128):
        if c <= n and n % c == 0:
            return c
    return n


def _dot(a, b, dims):
    return jax.lax.dot_general(a, b, dimension_numbers=(dims, ((), ())),
                               preferred_element_type=jnp.float32,
                               precision=_PREC)


def _layernorm(xb, g, b, eps=1e-5):
    mu = jnp.mean(xb, axis=-1, keepdims=True)
    var = jnp.mean((xb - mu) ** 2, axis=-1, keepdims=True)
    return (xb - mu) * jax.lax.rsqrt(var + eps) * g + b


def _mm_ln_body(x_ref, g_ref, b_ref, w_ref, bias_ref, o_ref):
    h = _layernorm(x_ref[...], g_ref[...], b_ref[...])
    o_ref[...] = (_dot(h.astype(jnp.bfloat16), w_ref[...].astype(jnp.bfloat16),
                       ((1,), (0,))) + bias_ref[...]).astype(o_ref.dtype)


def _mm_ln(x, g, b, w, bias, bt, bn, out_dtype=jnp.float32):
    t, d = x.shape
    n = w.shape[1]
    return pl.pallas_call(
        _mm_ln_body,
        grid=(t // bt, n // bn),
        in_specs=[
            pl.BlockSpec((bt, d), lambda i, j: (i, 0)),
            pl.BlockSpec((1, d), lambda i, j: (0, 0)),
            pl.BlockSpec((1, d), lambda i, j: (0, 0)),
            pl.BlockSpec((d, bn), lambda i, j: (0, j)),
            pl.BlockSpec((1, bn), lambda i, j: (0, j)),
        ],
        out_specs=pl.BlockSpec((bt, bn), lambda i, j: (i, j)),
        out_shape=jax.ShapeDtypeStruct((t, n), out_dtype),
    )(x, g.reshape(1, d), b.reshape(1, d), w, bias.reshape(1, n))


def _mm_res_body(a_ref, w_ref, bias_ref, r_ref, o_ref):
    o_ref[...] = (_dot(a_ref[...].astype(jnp.bfloat16),
                       w_ref[...].astype(jnp.bfloat16), ((1,), (0,)))
                  + bias_ref[...] + r_ref[...])


def _mm_res(a, w, bias, r, bt, bn):
    t, d = a.shape
    n = w.shape[1]
    return pl.pallas_call(
        _mm_res_body,
        grid=(t // bt, n // bn),
        in_specs=[
            pl.BlockSpec((bt, d), lambda i, j: (i, 0)),
            pl.BlockSpec((d, bn), lambda i, j: (0, j)),
            pl.BlockSpec((1, bn), lambda i, j: (0, j)),
            pl.BlockSpec((bt, bn), lambda i, j: (i, j)),
        ],
        out_specs=pl.BlockSpec((bt, bn), lambda i, j: (i, j)),
        out_shape=jax.ShapeDtypeStruct((t, n), jnp.float32),
    )(a, w, bias.reshape(1, n), r)


def _attn_body(q_ref, k_ref, v_ref, o_ref, *, scale, hd):
    q2 = q_ref[0]
    k2 = k_ref[0]
    v2 = v_ref[0]
    outs = []
    for hh in range(q2.shape[1] // hd):
        qh = q2[:, hh * hd:(hh + 1) * hd]
        kh = k2[:, hh * hd:(hh + 1) * hd]
        vh = v2[:, hh * hd:(hh + 1) * hd]
        s = _dot(qh, kh, ((1,), (1,))) * scale
        m = jnp.max(s, axis=-1, keepdims=True)
        p = jnp.exp(s - m)
        p = p / jnp.sum(p, axis=-1, keepdims=True)
        outs.append(_dot(p.astype(jnp.bfloat16), vh, ((1,), (0,))))
    o_ref[0] = jnp.concatenate(outs, axis=1).astype(jnp.bfloat16)


def _attention(qkv, n_h, hd, bq):
    b, s, n3 = qkv.shape
    d = n3 // 3
    hblk = 128 // hd
    scale = 1.0 / (hd ** 0.5)
    nh2 = n_h // hblk
    return pl.pallas_call(
        functools.partial(_attn_body, scale=scale, hd=hd),
        grid=(b, nh2, s // bq),
        in_specs=[
            pl.BlockSpec((1, bq, 128), lambda bb, h2, i: (bb, i, h2)),
            pl.BlockSpec((1, s, 128), lambda bb, h2, i: (bb, 0, nh2 + h2)),
            pl.BlockSpec((1, s, 128), lambda bb, h2, i: (bb, 0, 2 * nh2 + h2)),
        ],
        out_specs=pl.BlockSpec((1, bq, 128), lambda bb, h2, i: (bb, i, h2)),
        out_shape=jax.ShapeDtypeStruct((b, s, d), jnp.bfloat16),
    )(qkv, qkv, qkv)


def _ln_body(x_ref, g_ref, b_ref, o_ref, obf_ref):
    h = _layernorm(x_ref[...], g_ref[...], b_ref[...])
    o_ref[...] = h
    obf_ref[...] = h.astype(jnp.bfloat16)


def _ln_only(x, g, b, bt):
    t, d = x.shape
    return pl.pallas_call(
        _ln_body,
        grid=(t // bt,),
        in_specs=[
            pl.BlockSpec((bt, d), lambda i: (i, 0)),
            pl.BlockSpec((1, d), lambda i: (0, 0)),
            pl.BlockSpec((1, d), lambda i: (0, 0)),
        ],
        out_specs=[
            pl.BlockSpec((bt, d), lambda i: (i, 0)),
            pl.BlockSpec((bt, d), lambda i: (i, 0)),
        ],
        out_shape=[
            jax.ShapeDtypeStruct((t, d), jnp.float32),
            jax.ShapeDtypeStruct((t, d), jnp.bfloat16),
        ],
    )(x, g.reshape(1, d), b.reshape(1, d))


def _router_body(h_ref, wg_ref, bg_ref, disp_ref, bal_ref, ii_ref, gg_ref,
                 *, e):
    t = h_ref.shape[0]
    lanes = wg_ref.shape[1]
    logits = _dot(h_ref[...], wg_ref[...], ((1,), (0,))) + bg_ref[...]
    lane = jax.lax.broadcasted_iota(jnp.int32, (t, lanes), 1)
    neg = jnp.float32(-jnp.inf)
    logits = jnp.where(lane < e, logits, neg)
    i1 = jnp.argmax(logits, axis=1).astype(jnp.int32)[:, None]
    m1 = jnp.max(logits, axis=1, keepdims=True)
    masked = jnp.where(lane == i1, neg, logits)
    i2 = jnp.argmax(masked, axis=1).astype(jnp.int32)[:, None]
    m2 = jnp.max(masked, axis=1, keepdims=True)
    z = jnp.exp(m2 - m1)
    g1 = 1.0 / (1.0 + z)
    g2 = z / (1.0 + z)
    disp_ref[...] = (jnp.where(lane == i1, g1, 0.0)
                     + jnp.where(lane == i2, g2, 0.0))
    ii_ref[...] = jnp.where(lane == 0, i1, jnp.where(lane == 1, i2, 0))
    gg_ref[...] = jnp.where(lane == 0, g1, jnp.where(lane == 1, g2, 0.0))
    sm = jnp.exp(logits - m1)
    sm = sm / jnp.sum(sm, axis=1, keepdims=True)
    importance = jnp.mean(sm, axis=0)
    cnt = (jnp.where(lane == i1, 1.0, 0.0)
           + jnp.where(lane == i2, 1.0, 0.0))
    load = jnp.mean(cnt, axis=0)
    bal = jnp.float32(e) * jnp.sum(importance * load)
    bal_ref[...] = jnp.full(bal_ref.shape, bal, jnp.float32)


def _router(hf, wg, bg):
    t, d = hf.shape
    e = wg.shape[1]
    lanes = 128
    wg_pad = jnp.zeros((d, lanes), jnp.float32).at[:, :e].set(wg)
    bg_pad = jnp.zeros((1, lanes), jnp.float32).at[0, :e].set(bg)
    return pl.pallas_call(
        functools.partial(_router_body, e=e),
        grid=(1,),
        in_specs=[
            pl.BlockSpec((t, d), lambda i: (0, 0)),
            pl.BlockSpec((d, lanes), lambda i: (0, 0)),
            pl.BlockSpec((1, lanes), lambda i: (0, 0)),
        ],
        out_specs=[
            pl.BlockSpec((t, lanes), lambda i: (0, 0)),
            pl.BlockSpec((1, lanes), lambda i: (0, 0)),
            pl.BlockSpec((t, lanes), lambda i: (0, 0)),
            pl.BlockSpec((t, lanes), lambda i: (0, 0)),
        ],
        out_shape=[
            jax.ShapeDtypeStruct((t, lanes), jnp.float32),
            jax.ShapeDtypeStruct((1, lanes), jnp.float32),
            jax.ShapeDtypeStruct((t, lanes), jnp.int32),
            jax.ShapeDtypeStruct((t, lanes), jnp.float32),
        ],
    )(hf, wg_pad, bg_pad)


def _moe_body(h_ref, w1_ref, b1_ref, w2_ref, b2_ref, disp_ref,
              o_ref, acc_ref, *, n_e, n_f):
    e = pl.program_id(0)
    f = pl.program_id(1)
    bt = h_ref.shape[0]
    lanes = disp_ref.shape[1]
    lane = jax.lax.broadcasted_iota(jnp.int32, (bt, lanes), 1)
    w = jnp.sum(jnp.where(lane == e, disp_ref[...], 0.0), axis=1,
                keepdims=True)

    @pl.when((e == 0) & (f == 0))
    def _init():
        acc_ref[...] = jnp.zeros_like(acc_ref)

    @pl.when(f == 0)
    def _bias2():
        acc_ref[...] += w * b2_ref[0]

    he = jnp.maximum(_dot(h_ref[...], w1_ref[0].astype(jnp.bfloat16),
                          ((1,), (0,))) + b1_ref[0], 0.0)
    acc_ref[...] += w * _dot(he.astype(jnp.bfloat16),
                             w2_ref[0].astype(jnp.bfloat16), ((1,), (0,)))

    @pl.when((e == n_e - 1) & (f == n_f - 1))
    def _out():
        o_ref[...] = acc_ref[...].astype(jnp.bfloat16)


def _moe_dense(hbf, w1, b1, w2, b2, disp, bf):
    t, d = hbf.shape
    n_e, _, ff = w1.shape
    n_f = ff // bf
    lanes = disp.shape[1]
    return pl.pallas_call(
        functools.partial(_moe_body, n_e=n_e, n_f=n_f),
        grid=(n_e, n_f),
        in_specs=[
            pl.BlockSpec((t, d), lambda e, f: (0, 0)),
            pl.BlockSpec((1, d, bf), lambda e, f: (e, 0, f)),
            pl.BlockSpec((1, 1, bf), lambda e, f: (e, 0, f)),
            pl.BlockSpec((1, bf, d), lambda e, f: (e, f, 0)),
            pl.BlockSpec((1, 1, d), lambda e, f: (e, 0, 0)),
            pl.BlockSpec((t, lanes), lambda e, f: (0, 0)),
        ],
        out_specs=pl.BlockSpec((t, d), lambda e, f: (0, 0)),
        out_shape=jax.ShapeDtypeStruct((t, d), jnp.bfloat16),
        scratch_shapes=[
            pltpu.VMEM((t, d), jnp.float32),
        ],
    )(hbf, w1, b1.reshape(n_e, 1, ff), w2, b2.reshape(n_e, 1, d), disp)


def _sc_gather(table, idx):
    n, w = table.shape
    bsz = idx.shape[0]
    nw = 32
    b_per_w = bsz // nw
    ch = min(b_per_w, 128)
    n_ch = b_per_w // ch
    mesh = plsc.VectorSubcoreMesh(core_axis_name="c", subcore_axis_name="s")

    @functools.partial(
        pl.kernel, mesh=mesh,
        out_type=jax.ShapeDtypeStruct((bsz, w), jnp.int32),
        scratch_types=[
            pltpu.VMEM((ch,), jnp.int32),
            pltpu.VMEM((ch, w), jnp.int32),
            pltpu.SemaphoreType.DMA,
        ],
    )
    def k(table_hbm, idx_hbm, out_hbm, idx_v, rows_v, sem):
        wid = jax.lax.axis_index("s") * 2 + jax.lax.axis_index("c")
        for ci in range(n_ch):
            base = wid * b_per_w + ci * ch
            pltpu.sync_copy(idx_hbm.at[pl.ds(base, ch)], idx_v)
            pltpu.async_copy(table_hbm.at[idx_v], rows_v, sem).wait()
            pltpu.sync_copy(rows_v, out_hbm.at[pl.ds(base, ch)])

    return k(table, idx)


def _gather_rows_bf16(table_bf, idx):
    nrow, d = table_bf.shape
    t32 = jax.lax.bitcast_convert_type(
        table_bf.reshape(nrow, d // 2, 2), jnp.int32)
    out32 = _sc_gather(t32, idx)
    return jax.lax.bitcast_convert_type(
        out32, jnp.bfloat16).reshape(idx.shape[0], d)


def _gmm_body(g_ref, h_ref, w1_ref, b1_ref, w2_ref, b2_ref, o_ref, acc_ref):
    f = pl.program_id(1)
    n_f = pl.num_programs(1)

    @pl.when(f == 0)
    def _init():
        acc_ref[...] = jnp.broadcast_to(b2_ref[0], acc_ref.shape)

    he = jnp.maximum(_dot(h_ref[...], w1_ref[0].astype(jnp.bfloat16),
                          ((1,), (0,))) + b1_ref[0], 0.0)
    acc_ref[...] += _dot(he.astype(jnp.bfloat16),
                         w2_ref[0].astype(jnp.bfloat16), ((1,), (0,)))

    @pl.when(f == n_f - 1)
    def _out():
        o_ref[...] = acc_ref[...].astype(jnp.bfloat16)


def _gmm(sorted_h, w1, b1, w2, b2, g_ids, bt, bf):
    c, d = sorted_h.shape
    n_e, _, ff = w1.shape
    grid_spec = pltpu.PrefetchScalarGridSpec(
        num_scalar_prefetch=1,
        grid=(c // bt, ff // bf),
        in_specs=[
            pl.BlockSpec((bt, d), lambda i, f, g: (i, 0)),
            pl.BlockSpec((1, d, bf), lambda i, f, g: (g[i], 0, f)),
            pl.BlockSpec((1, 1, bf), lambda i, f, g: (g[i], 0, f)),
            pl.BlockSpec((1, bf, d), lambda i, f, g: (g[i], f, 0)),
            pl.BlockSpec((1, 1, d), lambda i, f, g: (g[i], 0, 0)),
        ],
        out_specs=pl.BlockSpec((bt, d), lambda i, f, g: (i, 0)),
        scratch_shapes=[pltpu.VMEM((bt, d), jnp.float32)],
    )
    return pl.pallas_call(
        _gmm_body,
        grid_spec=grid_spec,
        out_shape=jax.ShapeDtypeStruct((c, d), jnp.bfloat16),
    )(g_ids, sorted_h, w1, b1.reshape(n_e, 1, ff), w2,
      b2.reshape(n_e, 1, d))


def _combine_body(x2_ref, y1_ref, y2_ref, gg_ref, o_ref):
    g1 = gg_ref[...][:, 0:1]
    g2 = gg_ref[...][:, 1:2]
    o_ref[...] = (x2_ref[...]
                  + g1 * y1_ref[...].astype(jnp.float32)
                  + g2 * y2_ref[...].astype(jnp.float32))


def _combine(x2, y1, y2, gg, bt):
    t, d = x2.shape
    lanes = gg.shape[1]
    return pl.pallas_call(
        _combine_body,
        grid=(t // bt,),
        in_specs=[
            pl.BlockSpec((bt, d), lambda i: (i, 0)),
            pl.BlockSpec((bt, d), lambda i: (i, 0)),
            pl.BlockSpec((bt, d), lambda i: (i, 0)),
            pl.BlockSpec((bt, lanes), lambda i: (i, 0)),
        ],
        out_specs=pl.BlockSpec((bt, d), lambda i: (i, 0)),
        out_shape=jax.ShapeDtypeStruct((t, d), jnp.float32),
    )(x2, y1, y2, gg)


def _add_body(a_ref, b_ref, o_ref):
    o_ref[...] = a_ref[...] + b_ref[...].astype(jnp.float32)


def _residual_add(a, b, bt):
    t, d = a.shape
    return pl.pallas_call(
        _add_body,
        grid=(t // bt,),
        in_specs=[
            pl.BlockSpec((bt, d), lambda i: (i, 0)),
            pl.BlockSpec((bt, d), lambda i: (i, 0)),
        ],
        out_specs=pl.BlockSpec((bt, d), lambda i: (i, 0)),
        out_shape=jax.ShapeDtypeStruct((t, d), jnp.float32),
    )(a, b)


def kernel(x, ln1_g, ln1_b, Wq, bq, Wk, bk, Wv, bv, Wo, bo, ln2_g, ln2_b,
           Wg, bg, W1, b1, W2, b2):
    b, s, d = x.shape
    n_h = 16
    hd = d // n_h
    t = b * s
    xf = x.reshape(t, d)

    bt = _pick(t, 2048)
    wqkv = jnp.concatenate([Wq, Wk, Wv], axis=1)
    bqkv = jnp.concatenate([bq, bk, bv], axis=0)
    qkv = _mm_ln(xf, ln1_g, ln1_b, wqkv, bqkv, bt, _pick(3 * d, 512),
                 out_dtype=jnp.bfloat16)

    o = _attention(qkv.reshape(b, s, 3 * d), n_h, hd, _pick(s, 1024))
    o = o.reshape(t, d)

    x2 = _mm_res(o, Wo, bo, xf, bt, _pick(d, 512))

    hf, hbf = _ln_only(x2, ln2_g, ln2_b, bt)
    disp, bal, ii, gg = _router(hf, Wg, bg)

    n_e = Wg.shape[1]
    bt_g = 512
    c_max = t * 2 + n_e * bt_g
    i1 = ii[:, 0]
    i2 = ii[:, 1]
    oneh = (jax.nn.one_hot(i1, n_e, dtype=jnp.int32)
            + jax.nn.one_hot(i2, n_e, dtype=jnp.int32))
    excl = jnp.cumsum(oneh, axis=0) - oneh
    rank1 = jnp.take_along_axis(excl, i1[:, None], 1)[:, 0]
    rank2 = jnp.take_along_axis(excl, i2[:, None], 1)[:, 0]
    cnts = jnp.sum(oneh, axis=0)
    pad_cnt = ((cnts + bt_g - 1) // bt_g) * bt_g
    pad_off = jnp.cumsum(pad_cnt) - pad_cnt
    dest1 = pad_off[i1] + rank1
    dest2 = pad_off[i2] + rank2
    pad_end = pad_off + pad_cnt
    blk_start = jnp.arange(c_max // bt_g, dtype=jnp.int32) * bt_g
    g_ids = jnp.clip(jnp.sum((blk_start[:, None] >= pad_end[None, :])
                             .astype(jnp.int32), axis=1), 0, n_e - 1)
    tok = jnp.arange(t, dtype=jnp.int32)
    src = (jnp.zeros((c_max,), jnp.int32).at[dest1].set(tok)
           .at[dest2].set(tok))

    sorted_h = _gather_rows_bf16(hbf, src)
    y_sorted = _gmm(sorted_h, W1, b1, W2, b2, g_ids, bt_g,
                    _pick(W1.shape[2], 1024))
    y1 = _gather_rows_bf16(y_sorted, dest1)
    y2 = _gather_rows_bf16(y_sorted, dest2)
    out = _combine(x2, y1, y2, gg, _pick(t, 512))

    return (out.reshape(b, s, d), bal[0, 0])


# speedup vs baseline: 1.8294x; 1.8294x over previous
import functools

import jax
import jax.numpy as jnp
from jax.experimental import pallas as pl
from jax.experimental.pallas import tpu as pltpu
from jax.experimental.pallas import tpu_sc as plsc

_PREC = jax.lax.Precision.DEFAULT


def _pick(n, pref):
    for c in (pref, 512, 256, 128):
        if c <= n and n % c == 0:
            return c
    return n


def _dot(a, b, dims):
    return jax.lax.dot_general(a, b, dimension_numbers=(dims, ((), ())),
                               preferred_element_type=jnp.float32,
                               precision=_PREC)


def _layernorm(xb, g, b, eps=1e-5):
    mu = jnp.mean(xb, axis=-1, keepdims=True)
    var = jnp.mean((xb - mu) ** 2, axis=-1, keepdims=True)
    return (xb - mu) * jax.lax.rsqrt(var + eps) * g + b


def _mm_ln_body(x_ref, g_ref, b_ref, w_ref, bias_ref, o_ref):
    h = _layernorm(x_ref[...], g_ref[...], b_ref[...])
    o_ref[...] = (_dot(h.astype(jnp.bfloat16), w_ref[...].astype(jnp.bfloat16),
                       ((1,), (0,))) + bias_ref[...]).astype(o_ref.dtype)


def _mm_ln(x, g, b, w, bias, bt, bn, out_dtype=jnp.float32):
    t, d = x.shape
    n = w.shape[1]
    return pl.pallas_call(
        _mm_ln_body,
        grid=(t // bt, n // bn),
        in_specs=[
            pl.BlockSpec((bt, d), lambda i, j: (i, 0)),
            pl.BlockSpec((1, d), lambda i, j: (0, 0)),
            pl.BlockSpec((1, d), lambda i, j: (0, 0)),
            pl.BlockSpec((d, bn), lambda i, j: (0, j)),
            pl.BlockSpec((1, bn), lambda i, j: (0, j)),
        ],
        out_specs=pl.BlockSpec((bt, bn), lambda i, j: (i, j)),
        out_shape=jax.ShapeDtypeStruct((t, n), out_dtype),
    )(x, g.reshape(1, d), b.reshape(1, d), w, bias.reshape(1, n))


def _mm_res_body(a_ref, w_ref, bias_ref, r_ref, o_ref):
    o_ref[...] = (_dot(a_ref[...].astype(jnp.bfloat16),
                       w_ref[...].astype(jnp.bfloat16), ((1,), (0,)))
                  + bias_ref[...] + r_ref[...])


def _mm_res(a, w, bias, r, bt, bn):
    t, d = a.shape
    n = w.shape[1]
    return pl.pallas_call(
        _mm_res_body,
        grid=(t // bt, n // bn),
        in_specs=[
            pl.BlockSpec((bt, d), lambda i, j: (i, 0)),
            pl.BlockSpec((d, bn), lambda i, j: (0, j)),
            pl.BlockSpec((1, bn), lambda i, j: (0, j)),
            pl.BlockSpec((bt, bn), lambda i, j: (i, j)),
        ],
        out_specs=pl.BlockSpec((bt, bn), lambda i, j: (i, j)),
        out_shape=jax.ShapeDtypeStruct((t, n), jnp.float32),
    )(a, w, bias.reshape(1, n), r)


def _attn_body(q_ref, k_ref, v_ref, o_ref, *, scale, hd):
    q2 = q_ref[0]
    k2 = k_ref[0]
    v2 = v_ref[0]
    outs = []
    for hh in range(q2.shape[1] // hd):
        qh = q2[:, hh * hd:(hh + 1) * hd]
        kh = k2[:, hh * hd:(hh + 1) * hd]
        vh = v2[:, hh * hd:(hh + 1) * hd]
        s = _dot(qh, kh, ((1,), (1,))) * scale
        m = jnp.max(s, axis=-1, keepdims=True)
        p = jnp.exp(s - m)
        p = p / jnp.sum(p, axis=-1, keepdims=True)
        outs.append(_dot(p.astype(jnp.bfloat16), vh, ((1,), (0,))))
    o_ref[0] = jnp.concatenate(outs, axis=1).astype(jnp.bfloat16)


def _attention(qkv, n_h, hd, bq):
    b, s, n3 = qkv.shape
    d = n3 // 3
    hblk = 128 // hd
    scale = 1.0 / (hd ** 0.5)
    nh2 = n_h // hblk
    return pl.pallas_call(
        functools.partial(_attn_body, scale=scale, hd=hd),
        grid=(b, nh2, s // bq),
        in_specs=[
            pl.BlockSpec((1, bq, 128), lambda bb, h2, i: (bb, i, h2)),
            pl.BlockSpec((1, s, 128), lambda bb, h2, i: (bb, 0, nh2 + h2)),
            pl.BlockSpec((1, s, 128), lambda bb, h2, i: (bb, 0, 2 * nh2 + h2)),
        ],
        out_specs=pl.BlockSpec((1, bq, 128), lambda bb, h2, i: (bb, i, h2)),
        out_shape=jax.ShapeDtypeStruct((b, s, d), jnp.bfloat16),
    )(qkv, qkv, qkv)


def _ln_body(x_ref, g_ref, b_ref, o_ref, obf_ref):
    h = _layernorm(x_ref[...], g_ref[...], b_ref[...])
    o_ref[...] = h
    obf_ref[...] = h.astype(jnp.bfloat16)


def _ln_only(x, g, b, bt):
    t, d = x.shape
    return pl.pallas_call(
        _ln_body,
        grid=(t // bt,),
        in_specs=[
            pl.BlockSpec((bt, d), lambda i: (i, 0)),
            pl.BlockSpec((1, d), lambda i: (0, 0)),
            pl.BlockSpec((1, d), lambda i: (0, 0)),
        ],
        out_specs=[
            pl.BlockSpec((bt, d), lambda i: (i, 0)),
            pl.BlockSpec((bt, d), lambda i: (i, 0)),
        ],
        out_shape=[
            jax.ShapeDtypeStruct((t, d), jnp.float32),
            jax.ShapeDtypeStruct((t, d), jnp.bfloat16),
        ],
    )(x, g.reshape(1, d), b.reshape(1, d))


def _router_body(h_ref, wg_ref, bg_ref, disp_ref, bal_ref, ii_ref, gg_ref,
                 *, e):
    t = h_ref.shape[0]
    lanes = wg_ref.shape[1]
    logits = _dot(h_ref[...], wg_ref[...], ((1,), (0,))) + bg_ref[...]
    lane = jax.lax.broadcasted_iota(jnp.int32, (t, lanes), 1)
    neg = jnp.float32(-jnp.inf)
    logits = jnp.where(lane < e, logits, neg)
    i1 = jnp.argmax(logits, axis=1).astype(jnp.int32)[:, None]
    m1 = jnp.max(logits, axis=1, keepdims=True)
    masked = jnp.where(lane == i1, neg, logits)
    i2 = jnp.argmax(masked, axis=1).astype(jnp.int32)[:, None]
    m2 = jnp.max(masked, axis=1, keepdims=True)
    z = jnp.exp(m2 - m1)
    g1 = 1.0 / (1.0 + z)
    g2 = z / (1.0 + z)
    disp_ref[...] = (jnp.where(lane == i1, g1, 0.0)
                     + jnp.where(lane == i2, g2, 0.0))
    ii_ref[...] = jnp.where(lane == 0, i1, jnp.where(lane == 1, i2, 0))
    gg_ref[...] = jnp.where(lane == 0, g1, jnp.where(lane == 1, g2, 0.0))
    sm = jnp.exp(logits - m1)
    sm = sm / jnp.sum(sm, axis=1, keepdims=True)
    importance = jnp.mean(sm, axis=0)
    cnt = (jnp.where(lane == i1, 1.0, 0.0)
           + jnp.where(lane == i2, 1.0, 0.0))
    load = jnp.mean(cnt, axis=0)
    bal = jnp.float32(e) * jnp.sum(importance * load)
    bal_ref[...] = jnp.full(bal_ref.shape, bal, jnp.float32)


def _router(hf, wg, bg):
    t, d = hf.shape
    e = wg.shape[1]
    lanes = 128
    wg_pad = jnp.zeros((d, lanes), jnp.float32).at[:, :e].set(wg)
    bg_pad = jnp.zeros((1, lanes), jnp.float32).at[0, :e].set(bg)
    return pl.pallas_call(
        functools.partial(_router_body, e=e),
        grid=(1,),
        in_specs=[
            pl.BlockSpec((t, d), lambda i: (0, 0)),
            pl.BlockSpec((d, lanes), lambda i: (0, 0)),
            pl.BlockSpec((1, lanes), lambda i: (0, 0)),
        ],
        out_specs=[
            pl.BlockSpec((t, lanes), lambda i: (0, 0)),
            pl.BlockSpec((1, lanes), lambda i: (0, 0)),
            pl.BlockSpec((t, lanes), lambda i: (0, 0)),
            pl.BlockSpec((t, lanes), lambda i: (0, 0)),
        ],
        out_shape=[
            jax.ShapeDtypeStruct((t, lanes), jnp.float32),
            jax.ShapeDtypeStruct((1, lanes), jnp.float32),
            jax.ShapeDtypeStruct((t, lanes), jnp.int32),
            jax.ShapeDtypeStruct((t, lanes), jnp.float32),
        ],
    )(hf, wg_pad, bg_pad)


def _moe_body(h_ref, w1_ref, b1_ref, w2_ref, b2_ref, disp_ref,
              o_ref, acc_ref, *, n_e, n_f):
    e = pl.program_id(0)
    f = pl.program_id(1)
    bt = h_ref.shape[0]
    lanes = disp_ref.shape[1]
    lane = jax.lax.broadcasted_iota(jnp.int32, (bt, lanes), 1)
    w = jnp.sum(jnp.where(lane == e, disp_ref[...], 0.0), axis=1,
                keepdims=True)

    @pl.when((e == 0) & (f == 0))
    def _init():
        acc_ref[...] = jnp.zeros_like(acc_ref)

    @pl.when(f == 0)
    def _bias2():
        acc_ref[...] += w * b2_ref[0]

    he = jnp.maximum(_dot(h_ref[...], w1_ref[0].astype(jnp.bfloat16),
                          ((1,), (0,))) + b1_ref[0], 0.0)
    acc_ref[...] += w * _dot(he.astype(jnp.bfloat16),
                             w2_ref[0].astype(jnp.bfloat16), ((1,), (0,)))

    @pl.when((e == n_e - 1) & (f == n_f - 1))
    def _out():
        o_ref[...] = acc_ref[...].astype(jnp.bfloat16)


def _moe_dense(hbf, w1, b1, w2, b2, disp, bf):
    t, d = hbf.shape
    n_e, _, ff = w1.shape
    n_f = ff // bf
    lanes = disp.shape[1]
    return pl.pallas_call(
        functools.partial(_moe_body, n_e=n_e, n_f=n_f),
        grid=(n_e, n_f),
        in_specs=[
            pl.BlockSpec((t, d), lambda e, f: (0, 0)),
            pl.BlockSpec((1, d, bf), lambda e, f: (e, 0, f)),
            pl.BlockSpec((1, 1, bf), lambda e, f: (e, 0, f)),
            pl.BlockSpec((1, bf, d), lambda e, f: (e, f, 0)),
            pl.BlockSpec((1, 1, d), lambda e, f: (e, 0, 0)),
            pl.BlockSpec((t, lanes), lambda e, f: (0, 0)),
        ],
        out_specs=pl.BlockSpec((t, d), lambda e, f: (0, 0)),
        out_shape=jax.ShapeDtypeStruct((t, d), jnp.bfloat16),
        scratch_shapes=[
            pltpu.VMEM((t, d), jnp.float32),
        ],
    )(hbf, w1, b1.reshape(n_e, 1, ff), w2, b2.reshape(n_e, 1, d), disp)


def _sc_gather(table, idx):
    n, w = table.shape
    bsz = idx.shape[0]
    nw = 32
    b_per_w = bsz // nw
    ch = min(b_per_w, 128)
    n_ch = b_per_w // ch
    mesh = plsc.VectorSubcoreMesh(core_axis_name="c", subcore_axis_name="s")

    @functools.partial(
        pl.kernel, mesh=mesh,
        out_type=jax.ShapeDtypeStruct((bsz, w), jnp.int32),
        scratch_types=[
            pltpu.VMEM((ch,), jnp.int32),
            pltpu.VMEM((ch, w), jnp.int32),
            pltpu.SemaphoreType.DMA,
        ],
    )
    def k(table_hbm, idx_hbm, out_hbm, idx_v, rows_v, sem):
        wid = jax.lax.axis_index("s") * 2 + jax.lax.axis_index("c")
        for ci in range(n_ch):
            base = wid * b_per_w + ci * ch
            pltpu.sync_copy(idx_hbm.at[pl.ds(base, ch)], idx_v)
            pltpu.async_copy(table_hbm.at[idx_v], rows_v, sem).wait()
            pltpu.sync_copy(rows_v, out_hbm.at[pl.ds(base, ch)])

    return k(table, idx)


def _gather_rows_bf16(table_bf, idx):
    nrow, d = table_bf.shape
    t32 = jax.lax.bitcast_convert_type(
        table_bf.reshape(nrow, d // 2, 2), jnp.int32)
    out32 = _sc_gather(t32, idx)
    return jax.lax.bitcast_convert_type(
        out32, jnp.bfloat16).reshape(idx.shape[0], d)


def _gmm_body(g_ref, h_ref, w1_ref, b1_ref, w2_ref, b2_ref, o_ref, acc_ref):
    f = pl.program_id(1)
    n_f = pl.num_programs(1)

    @pl.when(f == 0)
    def _init():
        acc_ref[...] = jnp.broadcast_to(b2_ref[0], acc_ref.shape)

    he = jnp.maximum(_dot(h_ref[...], w1_ref[0].astype(jnp.bfloat16),
                          ((1,), (0,))) + b1_ref[0], 0.0)
    acc_ref[...] += _dot(he.astype(jnp.bfloat16),
                         w2_ref[0].astype(jnp.bfloat16), ((1,), (0,)))

    @pl.when(f == n_f - 1)
    def _out():
        o_ref[...] = acc_ref[...].astype(jnp.bfloat16)


def _gmm(sorted_h, w1, b1, w2, b2, g_ids, bt, bf):
    c, d = sorted_h.shape
    n_e, _, ff = w1.shape
    grid_spec = pltpu.PrefetchScalarGridSpec(
        num_scalar_prefetch=1,
        grid=(c // bt, ff // bf),
        in_specs=[
            pl.BlockSpec((bt, d), lambda i, f, g: (i, 0)),
            pl.BlockSpec((1, d, bf), lambda i, f, g: (g[i], 0, f)),
            pl.BlockSpec((1, 1, bf), lambda i, f, g: (g[i], 0, f)),
            pl.BlockSpec((1, bf, d), lambda i, f, g: (g[i], f, 0)),
            pl.BlockSpec((1, 1, d), lambda i, f, g: (g[i], 0, 0)),
        ],
        out_specs=pl.BlockSpec((bt, d), lambda i, f, g: (i, 0)),
        scratch_shapes=[pltpu.VMEM((bt, d), jnp.float32)],
    )
    return pl.pallas_call(
        _gmm_body,
        grid_spec=grid_spec,
        out_shape=jax.ShapeDtypeStruct((c, d), jnp.bfloat16),
    )(g_ids, sorted_h, w1, b1.reshape(n_e, 1, ff), w2,
      b2.reshape(n_e, 1, d))


def _combine_body(x2_ref, y1_ref, y2_ref, gg_ref, o_ref):
    g1 = gg_ref[...][:, 0:1]
    g2 = gg_ref[...][:, 1:2]
    o_ref[...] = (x2_ref[...]
                  + g1 * y1_ref[...].astype(jnp.float32)
                  + g2 * y2_ref[...].astype(jnp.float32))


def _combine(x2, y1, y2, gg, bt):
    t, d = x2.shape
    lanes = gg.shape[1]
    return pl.pallas_call(
        _combine_body,
        grid=(t // bt,),
        in_specs=[
            pl.BlockSpec((bt, d), lambda i: (i, 0)),
            pl.BlockSpec((bt, d), lambda i: (i, 0)),
            pl.BlockSpec((bt, d), lambda i: (i, 0)),
            pl.BlockSpec((bt, lanes), lambda i: (i, 0)),
        ],
        out_specs=pl.BlockSpec((bt, d), lambda i: (i, 0)),
        out_shape=jax.ShapeDtypeStruct((t, d), jnp.float32),
    )(x2, y1, y2, gg)


def _add_body(a_ref, b_ref, o_ref):
    o_ref[...] = a_ref[...] + b_ref[...].astype(jnp.float32)


def _residual_add(a, b, bt):
    t, d = a.shape
    return pl.pallas_call(
        _add_body,
        grid=(t // bt,),
        in_specs=[
            pl.BlockSpec((bt, d), lambda i: (i, 0)),
            pl.BlockSpec((bt, d), lambda i: (i, 0)),
        ],
        out_specs=pl.BlockSpec((bt, d), lambda i: (i, 0)),
        out_shape=jax.ShapeDtypeStruct((t, d), jnp.float32),
    )(a, b)


def kernel(x, ln1_g, ln1_b, Wq, bq, Wk, bk, Wv, bv, Wo, bo, ln2_g, ln2_b,
           Wg, bg, W1, b1, W2, b2):
    b, s, d = x.shape
    n_h = 16
    hd = d // n_h
    t = b * s
    xf = x.reshape(t, d)

    bt = _pick(t, 2048)
    wqkv = jnp.concatenate([Wq, Wk, Wv], axis=1)
    bqkv = jnp.concatenate([bq, bk, bv], axis=0)
    qkv = _mm_ln(xf, ln1_g, ln1_b, wqkv, bqkv, bt, _pick(3 * d, 512),
                 out_dtype=jnp.bfloat16)

    o = _attention(qkv.reshape(b, s, 3 * d), n_h, hd, _pick(s, 1024))
    o = o.reshape(t, d)

    x2 = _mm_res(o, Wo, bo, xf, bt, _pick(d, 512))

    hf, hbf = _ln_only(x2, ln2_g, ln2_b, bt)
    disp, bal, ii, gg = _router(hf, Wg, bg)

    n_e = Wg.shape[1]
    bt_g = 512
    c_max = t * 2 + n_e * bt_g
    i1 = ii[:, 0]
    i2 = ii[:, 1]
    oneh = (jax.nn.one_hot(i1, n_e, dtype=jnp.int32)
            + jax.nn.one_hot(i2, n_e, dtype=jnp.int32))
    excl = jnp.cumsum(oneh, axis=0) - oneh
    rank1 = jnp.take_along_axis(excl, i1[:, None], 1)[:, 0]
    rank2 = jnp.take_along_axis(excl, i2[:, None], 1)[:, 0]
    cnts = jnp.sum(oneh, axis=0)
    pad_cnt = ((cnts + bt_g - 1) // bt_g) * bt_g
    pad_off = jnp.cumsum(pad_cnt) - pad_cnt
    dest1 = pad_off[i1] + rank1
    dest2 = pad_off[i2] + rank2
    pad_end = pad_off + pad_cnt
    blk_start = jnp.arange(c_max // bt_g, dtype=jnp.int32) * bt_g
    g_ids = jnp.clip(jnp.sum((blk_start[:, None] >= pad_end[None, :])
                             .astype(jnp.int32), axis=1), 0, n_e - 1)
    tok = jnp.arange(t, dtype=jnp.int32)
    src = (jnp.zeros((c_max,), jnp.int32).at[dest1].set(tok)
           .at[dest2].set(tok))

    sorted_h = jnp.take(hbf, src, axis=0)
    y_sorted = _gmm(sorted_h, W1, b1, W2, b2, g_ids, bt_g,
                    _pick(W1.shape[2], 1024))
    y1 = jnp.take(y_sorted, dest1, axis=0)
    y2 = jnp.take(y_sorted, dest2, axis=0)
    out = _combine(x2, y1, y2, gg, _pick(t, 512))

    return (out.reshape(b, s, d), bal[0, 0])
